# baseline (speedup 1.0000x reference)
import functools

import jax
import jax.numpy as jnp
from jax import lax
from jax.experimental import pallas as pl
from jax.experimental.pallas import tpu as pltpu
from jax.experimental.pallas import tpu_sc as plsc

N_NODES = 10000
D = 128
DH = D // 2
L = 16
NC, NS = 2, 16
NW = NC * NS
CH = 128
CPT = 80
EPT = CH * CPT
E_PAD = NW * EPT
HC0 = 56
HC1 = 24
HCMX = max(HC0, HC1)
ACC_ROWS = 10112
RPS = ACC_ROWS // NS
DEG_W = 8

_MESH = plsc.VectorSubcoreMesh(
    core_axis_name="c", subcore_axis_name="s", num_cores=NC, num_subcores=NS)
_MESH1 = plsc.VectorSubcoreMesh(
    core_axis_name="c", subcore_axis_name="s", num_cores=1, num_subcores=NS)


@functools.partial(
    pl.kernel,
    out_type=jax.ShapeDtypeStruct((NW, ACC_ROWS), jnp.int32),
    mesh=_MESH,
    scratch_types=[
        pltpu.VMEM((CPT, CH), jnp.int32),
        pltpu.VMEM((ACC_ROWS,), jnp.int32),
    ],
    compiler_params=pltpu.CompilerParams(needs_layout_passes=False),
)
def _sc_hist(dst_hbm, out_hbm, idx_v, hist_v):
    c = lax.axis_index("c")
    s = lax.axis_index("s")
    wid = s * NC + c

    @pl.loop(0, ACC_ROWS // L)
    def _(i):
        hist_v[pl.ds(i * L, L)] = jnp.zeros((L,), jnp.int32)

    pltpu.sync_copy(dst_hbm.at[wid], idx_v)

    @pl.loop(0, CPT)
    def _(j):
        for k in range(CH // L):
            d16 = idx_v[j, pl.ds(k * L, L)]
            cnt, last = plsc.scan_count(d16)
            plsc.addupdate_scatter(hist_v, [d16], cnt, mask=last)

    pltpu.sync_copy(hist_v, out_hbm.at[wid])


@functools.partial(
    pl.kernel,
    out_type=jax.ShapeDtypeStruct((NC, ACC_ROWS, D), jnp.float32),
    mesh=_MESH,
    scratch_types=[
        pltpu.VMEM((HCMX, CH), jnp.int32),
        pltpu.VMEM((HCMX, CH), jnp.int32),
        pltpu.VMEM((2 * CH, D), jnp.float32),
        pltpu.VMEM_SHARED((ACC_ROWS, D), jnp.float32),
        pltpu.SemaphoreType.DMA,
    ],
)
def _sc_gather_scatter(g_hbm, src_hbm, dst_hbm, zeros_hbm, out_hbm,
                       src_v, dst_v, rows2, acc_sh, sga):
    c = lax.axis_index("c")
    s = lax.axis_index("s")

    def scatter(j, off):
        pltpu.sync_copy(rows2.at[pl.ds(off, CH)],
                        acc_sh.at[dst_v.at[j]], add=True)

    @pl.loop(0, RPS // CH)
    def _(k):
        pltpu.sync_copy(zeros_hbm, acc_sh.at[pl.ds(s * RPS + k * CH, CH)])

    if RPS % CH:
        pltpu.sync_copy(
            zeros_hbm.at[pl.ds(0, RPS % CH)],
            acc_sh.at[pl.ds(s * RPS + (RPS // CH) * CH, RPS % CH)])

    plsc.subcore_barrier()

    hc = jnp.where(c == 0, HC0, HC1)
    for half in range(2):
        pltpu.sync_copy(src_hbm.at[c, s, pl.ds(half * hc, HCMX)], src_v)
        pltpu.sync_copy(dst_hbm.at[c, s, pl.ds(half * hc, HCMX)], dst_v)

        @pl.loop(0, hc + 1)
        def _(t):
            @pl.when(t < hc)
            def _():
                pltpu.async_copy(g_hbm.at[src_v.at[t]],
                                 rows2.at[pl.ds((t % 2) * CH, CH)], sga)

            @pl.when(t > 0)
            def _():
                scatter(t - 1, ((t - 1) % 2) * CH)

            @pl.when(t < hc)
            def _():
                pltpu.make_async_copy(g_hbm.at[src_v.at[t]],
                                      rows2.at[pl.ds((t % 2) * CH, CH)],
                                      sga).wait()

    plsc.subcore_barrier()
    pltpu.sync_copy(acc_sh.at[pl.ds(s * RPS, RPS)],
                    out_hbm.at[c, pl.ds(s * RPS, RPS)])


def _tc_pre_body(x_ref, w1_ref, hist_ref, g_ref, dinv_ref):
    deg = jnp.sum(hist_ref[...].astype(jnp.float32), axis=1, keepdims=True) + 1.0
    dinv = lax.rsqrt(deg)
    dinv_ref[...] = dinv
    h = jnp.dot(x_ref[...], w1_ref[...], preferred_element_type=jnp.float32)
    g_ref[...] = h * dinv[:N_NODES]


_tc_pre = pl.pallas_call(
    _tc_pre_body,
    out_shape=[
        jax.ShapeDtypeStruct((N_NODES, D), jnp.float32),
        jax.ShapeDtypeStruct((ACC_ROWS, 1), jnp.float32),
    ],
)


def _tc_post_body(p_ref, g_ref, dinv_ref, b1_ref, w2_ref, b2_ref, o_ref):
    acc = p_ref[0, :N_NODES, :] + p_ref[1, :N_NODES, :] + g_ref[...]
    h1 = jnp.maximum(acc * dinv_ref[:N_NODES] + b1_ref[...], 0.0)
    o_ref[...] = (jnp.dot(h1, w2_ref[...], preferred_element_type=jnp.float32)
                  + b2_ref[...])


_tc_post = pl.pallas_call(
    _tc_post_body,
    out_shape=jax.ShapeDtypeStruct((N_NODES, D), jnp.float32),
)


def kernel(x, edge_index, W1, b1, W2, b2):
    ei = edge_index.astype(jnp.int32)
    n_edges = ei.shape[1]
    pad = E_PAD - n_edges
    src = jnp.concatenate([ei[0], jnp.zeros((pad,), jnp.int32)])
    dst = jnp.concatenate([ei[1], jnp.full((pad,), N_NODES, jnp.int32)])
    src3 = src.reshape(NW, CPT, CH)
    dst3 = dst.reshape(NW, CPT, CH)
    zeros_d = jnp.zeros((CH, D), jnp.float32)

    def split_core(v):
        n0 = NS * 2 * HC0 * CH
        c0 = v[:n0].reshape(NS, 2 * HC0, CH)
        c1 = v[n0:].reshape(NS, 2 * HC1, CH)
        rows = 2 * HCMX
        c0p = jnp.pad(c0, ((0, 0), (0, rows - 2 * HC0), (0, 0)))
        c1p = jnp.pad(c1, ((0, 0), (0, rows - 2 * HC1), (0, 0)))
        return jnp.stack([c0p, c1p])

    src_g = split_core(src)
    dst_g = split_core(dst)

    hist = _sc_hist(dst3)
    g, dinv = _tc_pre(x, W1, hist.T)
    parts = _sc_gather_scatter(g, src_g, dst_g, zeros_d)
    return _tc_post(parts, g, dinv, b1, W2, b2)


# speedup vs baseline: 1.2666x; 1.2666x over previous
import functools

import jax
import jax.numpy as jnp
from jax import lax
from jax.experimental import pallas as pl
from jax.experimental.pallas import tpu as pltpu
from jax.experimental.pallas import tpu_sc as plsc

N_NODES = 10000
D = 128
L = 16
NC, NS = 2, 16
NW = NC * NS
CH = 128
CPT = 80
EPT = CH * CPT
E_PAD = NW * EPT
HC0 = 64
HC1 = 16
HCMX = max(HC0, HC1)
ACC_ROWS = 10112
RPS = ACC_ROWS // NS

_MESH = plsc.VectorSubcoreMesh(
    core_axis_name="c", subcore_axis_name="s", num_cores=NC, num_subcores=NS)


@functools.partial(
    pl.kernel,
    out_type=jax.ShapeDtypeStruct((NW, ACC_ROWS), jnp.int32),
    mesh=_MESH,
    scratch_types=[
        pltpu.VMEM((CPT, CH), jnp.int32),
        pltpu.VMEM((ACC_ROWS,), jnp.int32),
    ],
    compiler_params=pltpu.CompilerParams(needs_layout_passes=False),
)
def _sc_hist(dst_hbm, out_hbm, idx_v, hist_v):
    c = lax.axis_index("c")
    s = lax.axis_index("s")
    wid = s * NC + c

    @pl.loop(0, ACC_ROWS // L)
    def _(i):
        hist_v[pl.ds(i * L, L)] = jnp.zeros((L,), jnp.int32)

    pltpu.sync_copy(dst_hbm.at[wid], idx_v)

    @pl.loop(0, CPT)
    def _(j):
        for k in range(CH // L):
            d16 = idx_v[j, pl.ds(k * L, L)]
            cnt, last = plsc.scan_count(d16)
            plsc.addupdate_scatter(hist_v, [d16], cnt, mask=last)

    pltpu.sync_copy(hist_v, out_hbm.at[wid])


@functools.partial(
    pl.kernel,
    out_type=jax.ShapeDtypeStruct((NC, ACC_ROWS, D), jnp.float32),
    mesh=_MESH,
    scratch_types=[
        pltpu.VMEM((HCMX, CH), jnp.int32),
        pltpu.VMEM((HCMX, CH), jnp.int32),
        pltpu.VMEM((2 * CH, D), jnp.float32),
        pltpu.VMEM_SHARED((ACC_ROWS, D), jnp.float32),
        pltpu.SemaphoreType.DMA,
    ],
)
def _sc_gather_scatter(g_hbm, src_hbm, dst_hbm, zeros_hbm, out_hbm,
                       src_v, dst_v, rows2, acc_sh, sga):
    c = lax.axis_index("c")
    s = lax.axis_index("s")

    def scatter(j, off):
        pltpu.sync_copy(rows2.at[pl.ds(off, CH)],
                        acc_sh.at[dst_v.at[j]], add=True)

    @pl.loop(0, RPS // CH)
    def _(k):
        pltpu.sync_copy(zeros_hbm, acc_sh.at[pl.ds(s * RPS + k * CH, CH)])

    if RPS % CH:
        pltpu.sync_copy(
            zeros_hbm.at[pl.ds(0, RPS % CH)],
            acc_sh.at[pl.ds(s * RPS + (RPS // CH) * CH, RPS % CH)])

    plsc.subcore_barrier()

    hc = jnp.where(c == 0, HC0, HC1)
    for half in range(2):
        pltpu.sync_copy(src_hbm.at[c, s, pl.ds(half * hc, HCMX)], src_v)
        pltpu.sync_copy(dst_hbm.at[c, s, pl.ds(half * hc, HCMX)], dst_v)

        @pl.loop(0, hc + 1)
        def _(t):
            @pl.when(t < hc)
            def _():
                pltpu.async_copy(g_hbm.at[src_v.at[t]],
                                 rows2.at[pl.ds((t % 2) * CH, CH)], sga)

            @pl.when(t > 0)
            def _():
                scatter(t - 1, ((t - 1) % 2) * CH)

            @pl.when(t < hc)
            def _():
                pltpu.make_async_copy(g_hbm.at[src_v.at[t]],
                                      rows2.at[pl.ds((t % 2) * CH, CH)],
                                      sga).wait()

    plsc.subcore_barrier()
    pltpu.sync_copy(acc_sh.at[pl.ds(s * RPS, RPS)],
                    out_hbm.at[c, pl.ds(s * RPS, RPS)])


def _tc_pre_body(x_ref, w1_ref, hist_ref, g_ref, dinv_ref):
    deg = jnp.sum(hist_ref[...].astype(jnp.float32), axis=1, keepdims=True) + 1.0
    dinv = lax.rsqrt(deg)
    dinv_ref[...] = dinv
    h = jnp.dot(x_ref[...], w1_ref[...], preferred_element_type=jnp.float32)
    g_ref[...] = h * dinv[:N_NODES]


_tc_pre = pl.pallas_call(
    _tc_pre_body,
    out_shape=[
        jax.ShapeDtypeStruct((N_NODES, D), jnp.float32),
        jax.ShapeDtypeStruct((ACC_ROWS, 1), jnp.float32),
    ],
)


def _tc_post_body(p_ref, g_ref, dinv_ref, b1_ref, w2_ref, b2_ref, o_ref):
    acc = p_ref[0, :N_NODES, :] + p_ref[1, :N_NODES, :] + g_ref[...]
    h1 = jnp.maximum(acc * dinv_ref[:N_NODES] + b1_ref[...], 0.0)
    o_ref[...] = (jnp.dot(h1, w2_ref[...], preferred_element_type=jnp.float32)
                  + b2_ref[...])


_tc_post = pl.pallas_call(
    _tc_post_body,
    out_shape=jax.ShapeDtypeStruct((N_NODES, D), jnp.float32),
)


def kernel(x, edge_index, W1, b1, W2, b2):
    ei = edge_index.astype(jnp.int32)
    n_edges = ei.shape[1]
    pad = E_PAD - n_edges
    src = jnp.concatenate([ei[0], jnp.zeros((pad,), jnp.int32)])
    dst = jnp.concatenate([ei[1], jnp.full((pad,), N_NODES, jnp.int32)])
    src3 = src.reshape(NW, CPT, CH)
    dst3 = dst.reshape(NW, CPT, CH)
    zeros_d = jnp.zeros((CH, D), jnp.float32)

    def split_core(v):
        n0 = NS * 2 * HC0 * CH
        c0 = v[:n0].reshape(NS, 2 * HC0, CH)
        c1 = v[n0:].reshape(NS, 2 * HC1, CH)
        rows = 2 * HCMX
        c0p = jnp.pad(c0, ((0, 0), (0, rows - 2 * HC0), (0, 0)))
        c1p = jnp.pad(c1, ((0, 0), (0, rows - 2 * HC1), (0, 0)))
        return jnp.stack([c0p, c1p])

    src_g = split_core(src)
    dst_g = split_core(dst)

    hist = _sc_hist(dst3)
    g, dinv = _tc_pre(x, W1, hist.T)
    parts = _sc_gather_scatter(g, src_g, dst_g, zeros_d)
    return _tc_post(parts, g, dinv, b1, W2, b2)
